# baseline (speedup 1.0000x reference)
import jax
import jax.numpy as jnp
from jax.experimental import pallas as pl
from jax.experimental.pallas import tpu as pltpu

B, N, DMAX = 64, 2048, 64
D = 128
BB = 8

_W1 = 0
_W2 = 64
_C1 = 192
_C2 = 193
_G1 = 194
_BE1 = 195
_WROWS = 200


def _row(xt, t, w_ref, wpb_ref, cpb_ref):
    hc = jax.lax.dot_general(xt, w_ref[_W1:_W1 + DMAX, :],
                             (((0,), (0,)), ((), ())),
                             preferred_element_type=jnp.float32)
    hc = hc + w_ref[_C1, :]
    var = jnp.mean(hc * hc, axis=-1, keepdims=True)
    h = hc * jax.lax.rsqrt(var + 1e-5) * w_ref[_G1, :] + w_ref[_BE1, :]
    h = jnp.maximum(h, 0.0)
    poly = jnp.dot(h, w_ref[_W2:_W2 + D, :],
                   preferred_element_type=jnp.float32)
    poly = poly + w_ref[_C2, :]

    ptbx = jax.lax.dot_general(xt[:8, :], wpb_ref[:, :],
                               (((0,), (0,)), ((), ())),
                               preferred_element_type=jnp.float32) + cpb_ref[0, :]
    pt = ptbx[:, :D]
    bx = ptbx[:, D:]

    return jnp.where(t == 0, pt, jnp.where(t == 1, bx, poly))


def _body(xt_ref, t_ref, w_ref, wpb_ref, cpb_ref, o_ref):
    for b in range(BB):
        xt = xt_ref[b]
        t = t_ref[b, 0].reshape(N, 1)
        o_ref[b] = _row(xt, t, w_ref, wpb_ref, cpb_ref)


def kernel(prompts, prompt_types, Wp, bp, Wb, bb, W1, b1, g1, be1, W2, b2,
           type_emb):
    pT = jnp.swapaxes(prompts, 1, 2)
    t3 = prompt_types.reshape(B, 1, N)

    W1c = W1 - jnp.mean(W1, axis=1, keepdims=True)
    b1c = b1 - jnp.mean(b1)

    w = jnp.zeros((_WROWS, D), jnp.float32)
    w = w.at[_W1:_W1 + DMAX, :].set(W1c)
    w = w.at[_W2:_W2 + D, :].set(W2)
    w = w.at[_C1, :].set(b1c)
    w = w.at[_C2, :].set(b2 + type_emb[2])
    w = w.at[_G1, :].set(g1)
    w = w.at[_BE1, :].set(be1)

    wpb = jnp.zeros((8, 2 * D), jnp.float32)
    wpb = wpb.at[:2, :D].set(Wp)
    wpb = wpb.at[:4, D:].set(Wb)
    cpb = jnp.concatenate([bp + type_emb[0], bb + type_emb[1]]).reshape(1, 2 * D)

    out = pl.pallas_call(
        _body,
        grid=(B // BB,),
        in_specs=[
            pl.BlockSpec((BB, DMAX, N), lambda i: (i, 0, 0)),
            pl.BlockSpec((BB, 1, N), lambda i: (i, 0, 0)),
            pl.BlockSpec((_WROWS, D), lambda i: (0, 0)),
            pl.BlockSpec((8, 2 * D), lambda i: (0, 0)),
            pl.BlockSpec((1, 2 * D), lambda i: (0, 0)),
        ],
        out_specs=pl.BlockSpec((BB, N, D), lambda i: (i, 0, 0)),
        out_shape=jax.ShapeDtypeStruct((B, N, D), jnp.float32),
        compiler_params=pltpu.CompilerParams(
            dimension_semantics=("arbitrary",),
        ),
    )(pT, t3, w, wpb, cpb)
    return out


# speedup vs baseline: 1.1766x; 1.1766x over previous
import jax
import jax.numpy as jnp
from jax.experimental import pallas as pl
from jax.experimental.pallas import tpu as pltpu

B, N, DMAX = 64, 2048, 64
D = 128
BB = 8

_W1 = 0
_W2 = 64
_C1 = 192
_C2 = 193
_G1 = 194
_BE1 = 195
_WROWS = 200


def _row(x, t, w_ref, wpb_ref, cpb_ref):
    hc = jnp.dot(x, w_ref[_W1:_W1 + DMAX, :],
                 preferred_element_type=jnp.float32)
    hc = hc + w_ref[_C1, :]
    var = jnp.mean(hc * hc, axis=-1, keepdims=True)
    h = hc * jax.lax.rsqrt(var + 1e-5) * w_ref[_G1, :] + w_ref[_BE1, :]
    h = jnp.maximum(h, 0.0)
    poly = jnp.dot(h, w_ref[_W2:_W2 + D, :],
                   preferred_element_type=jnp.float32)
    poly = poly + w_ref[_C2, :]

    ptbx = jnp.dot(x[:, :8], wpb_ref[:, :],
                   preferred_element_type=jnp.float32) + cpb_ref[0, :]
    pt = ptbx[:, :D]
    bx = ptbx[:, D:]

    return jnp.where(t == 0, pt, jnp.where(t == 1, bx, poly))


def _body(xt_ref, t_ref, w_ref, wpb_ref, cpb_ref, o_ref):
    for b in range(BB):
        x = jnp.transpose(xt_ref[b])
        t = t_ref[b, 0].reshape(N, 1)
        o_ref[b] = _row(x, t, w_ref, wpb_ref, cpb_ref)


def kernel(prompts, prompt_types, Wp, bp, Wb, bb, W1, b1, g1, be1, W2, b2,
           type_emb):
    pT = jnp.swapaxes(prompts, 1, 2)
    t3 = prompt_types.reshape(B, 1, N)

    W1c = W1 - jnp.mean(W1, axis=1, keepdims=True)
    b1c = b1 - jnp.mean(b1)

    w = jnp.zeros((_WROWS, D), jnp.float32)
    w = w.at[_W1:_W1 + DMAX, :].set(W1c)
    w = w.at[_W2:_W2 + D, :].set(W2)
    w = w.at[_C1, :].set(b1c)
    w = w.at[_C2, :].set(b2 + type_emb[2])
    w = w.at[_G1, :].set(g1)
    w = w.at[_BE1, :].set(be1)

    wpb = jnp.zeros((8, 2 * D), jnp.float32)
    wpb = wpb.at[:2, :D].set(Wp)
    wpb = wpb.at[:4, D:].set(Wb)
    cpb = jnp.concatenate([bp + type_emb[0], bb + type_emb[1]]).reshape(1, 2 * D)

    out = pl.pallas_call(
        _body,
        grid=(B // BB,),
        in_specs=[
            pl.BlockSpec((BB, DMAX, N), lambda i: (i, 0, 0)),
            pl.BlockSpec((BB, 1, N), lambda i: (i, 0, 0)),
            pl.BlockSpec((_WROWS, D), lambda i: (0, 0)),
            pl.BlockSpec((8, 2 * D), lambda i: (0, 0)),
            pl.BlockSpec((1, 2 * D), lambda i: (0, 0)),
        ],
        out_specs=pl.BlockSpec((BB, N, D), lambda i: (i, 0, 0)),
        out_shape=jax.ShapeDtypeStruct((B, N, D), jnp.float32),
        compiler_params=pltpu.CompilerParams(
            dimension_semantics=("arbitrary",),
        ),
    )(pT, t3, w, wpb, cpb)
    return out
